# baseline (speedup 1.0000x reference)
import functools

import jax
import jax.numpy as jnp
from jax import lax
from jax.experimental import pallas as pl
from jax.experimental.pallas import tpu as pltpu
from jax.experimental.pallas import tpu_sc as plsc

N = 10000
D = 128
K = 10
NC = 2
NS = 16
NT = NC * NS
NP = 10240
RPT = NP // NT
RPS = NP // NS
CH = 128
VL = 16

_mesh = plsc.VectorSubcoreMesh(
    core_axis_name="c", subcore_axis_name="s", num_cores=NC, num_subcores=NS)
_params = pltpu.CompilerParams(
    needs_layout_passes=False, use_tc_tiling_on_sc=False)


def _wid():
    return lax.axis_index("c") * NS + lax.axis_index("s")


def _splat_i32(i):
    return jnp.full((VL,), i, dtype=jnp.int32)


def _rsqrt(d):
    i = plsc.bitcast(d, jnp.int32)
    y = plsc.bitcast(jnp.int32(0x5F3759DF) - (i >> 1), jnp.float32)
    for _ in range(3):
        y = y * (1.5 - 0.5 * d * y * y)
    return y


def _zero_rows(buf, nrows):
    z = jnp.zeros((VL,), jnp.float32)

    def body(i, c):
        for l in range(D // VL):
            buf[i, pl.ds(l * VL, VL)] = z
        return c

    lax.fori_loop(0, nrows, body, 0)


def _deg_body(nch, colp_h, degp_h, colv, degv):
    w = _wid()
    z = jnp.zeros((VL,), jnp.float32)

    def zbody(i, c):
        degv[pl.ds(i * VL, VL)] = z
        return c

    lax.fori_loop(0, NP // VL, zbody, 0)
    pltpu.sync_copy(colp_h.at[w, pl.ds(0, nch)], colv)
    ones = jnp.ones((VL,), jnp.float32)

    def cbody(j, c):
        for l in range(CH // VL):
            idx = colv[j, pl.ds(l * VL, VL)]
            plsc.addupdate_scatter(degv, [idx], ones)
        return c

    lax.fori_loop(0, nch, cbody, 0)
    pltpu.sync_copy(degv, degp_h.at[w])


def _make_deg_kernel(nch):
    return pl.kernel(
        functools.partial(_deg_body, nch),
        out_type=jax.ShapeDtypeStruct((NT, NP), jnp.float32),
        mesh=_mesh,
        compiler_params=_params,
        scratch_types=[
            pltpu.VMEM((nch, CH), jnp.int32),
            pltpu.VMEM((NP,), jnp.float32),
        ],
    )


def _norm_body(degp_h, xp_h, temp_h, dinv2_h, dm_h, g0_h, gg0_h,
               degs, dv, d2v, dmv, xbuf, tbuf, sp):
    w = _wid()
    base = w * RPT
    for t in range(NT):
        pltpu.async_copy(degp_h.at[t, pl.ds(base, RPT)], degs.at[t], sp)
    for t in range(NT):
        pltpu.make_async_copy(
            degp_h.at[t, pl.ds(base, RPT)], degs.at[t], sp).wait()

    def rbody(i, c):
        acc = jnp.ones((VL,), jnp.float32)
        for t in range(NT):
            acc = acc + degs[t, pl.ds(i * VL, VL)]
        y = _rsqrt(acc)
        dv[pl.ds(i * VL, VL)] = y
        d2v[pl.ds(i * VL, VL)] = y * y
        dmv[pl.ds(i * VL, VL)] = acc * y
        return c

    lax.fori_loop(0, RPT // VL, rbody, 0)
    pltpu.sync_copy(d2v, dinv2_h.at[pl.ds(base, RPT)])
    pltpu.sync_copy(dmv, dm_h.at[pl.ds(base, RPT)])
    pltpu.sync_copy(temp_h, tbuf)
    t0 = tbuf[pl.ds(0, VL)]

    nr = 64
    for cc in range(RPT // nr):
        o = base + cc * nr
        pltpu.sync_copy(xp_h.at[pl.ds(o, nr)], xbuf)

        def sbody(i, c, cc=cc):
            dvi = plsc.load_gather(dv, [_splat_i32(cc * nr + i)])
            for l in range(D // VL):
                s = pl.ds(l * VL, VL)
                xbuf[i, s] = xbuf[i, s] * dvi
            return c

        lax.fori_loop(0, nr, sbody, 0)
        pltpu.sync_copy(xbuf, g0_h.at[pl.ds(o, nr)])

        def t0body(i, c):
            for l in range(D // VL):
                s = pl.ds(l * VL, VL)
                xbuf[i, s] = xbuf[i, s] * t0
            return c

        lax.fori_loop(0, nr, t0body, 0)
        pltpu.sync_copy(xbuf, gg0_h.at[pl.ds(o, nr)])


_norm_kernel = pl.kernel(
    _norm_body,
    out_type=(
        jax.ShapeDtypeStruct((NP,), jnp.float32),
        jax.ShapeDtypeStruct((NP,), jnp.float32),
        jax.ShapeDtypeStruct((NP, D), jnp.float32),
        jax.ShapeDtypeStruct((NP, D), jnp.float32),
    ),
    mesh=_mesh,
    compiler_params=_params,
    scratch_types=[
        pltpu.VMEM((NT, RPT), jnp.float32),
        pltpu.VMEM((RPT,), jnp.float32),
        pltpu.VMEM((RPT,), jnp.float32),
        pltpu.VMEM((RPT,), jnp.float32),
        pltpu.VMEM((64, D), jnp.float32),
        pltpu.VMEM((VL,), jnp.float32),
        pltpu.SemaphoreType.DMA,
    ],
)


NB = 3
AR = 10016
ART = AR // NS


def _scatter_body(nch, g_h, rowp_h, colp_h, p_h,
                  acc, rb, rix, cix, sg, ss, si, sd):
    cid = lax.axis_index("c")
    sid = lax.axis_index("s")
    w = cid * NS + sid

    def idx_start(s, grp):
        pltpu.async_copy(rowp_h.at[w, pl.ds(grp * NB, NB)], rix[s], si[s])
        pltpu.async_copy(colp_h.at[w, pl.ds(grp * NB, NB)], cix[s], si[s])

    def idx_wait(s):
        pltpu.make_async_copy(rowp_h.at[w, pl.ds(0, NB)], rix[s], si[s]).wait()
        pltpu.make_async_copy(colp_h.at[w, pl.ds(0, NB)], cix[s], si[s]).wait()

    def gwait(b):
        pltpu.make_async_copy(g_h.at[pl.ds(0, CH)], rb[b], sg[b]).wait()

    def swait(b):
        pltpu.make_async_copy(g_h.at[pl.ds(0, CH)], rb[b], ss[b]).wait()

    ngrp = nch // NB
    idx_start(0, 0)
    _zero_rows(rb[0], CH)
    zb = ART // CH
    for z in range(zb):
        pltpu.sync_copy(rb[0], acc.at[pl.ds(sid * ART + z * CH, CH)])
    pltpu.sync_copy(rb[0].at[pl.ds(0, ART - zb * CH)],
                    acc.at[pl.ds(sid * ART + zb * CH, ART - zb * CH)])
    idx_wait(0)
    for b in range(NB):
        pltpu.async_copy(g_h.at[rix[0].at[b]], rb[b], sg[b])
    idx_start(1, 1)
    plsc.subcore_barrier()

    def group(gg, s_cur, s_nxt):
        for b in range(NB):
            gwait(b)
            pltpu.async_copy(rb[b], acc.at[cix[s_cur].at[b]], ss[b], add=True)
        idx_wait(s_nxt)
        for b in range(NB):
            swait(b)
            pltpu.async_copy(g_h.at[rix[s_nxt].at[b]], rb[b], sg[b])
        idx_start(s_cur, gg + 2)

    def ebody(m2, c):
        group(2 * m2, 0, 1)
        group(2 * m2 + 1, 1, 0)
        return c

    lax.fori_loop(0, (ngrp - 1) // 2, ebody, 0)
    idx_wait(1)
    for b in range(NB):
        gwait(b)
        pltpu.async_copy(rb[b], acc.at[cix[0].at[b]], ss[b], add=True)
    for b in range(NB):
        swait(b)
    plsc.subcore_barrier()
    dumps = [pl.ds(sid * ART + z * CH, CH) for z in range(zb)]
    dumps.append(pl.ds(sid * ART + zb * CH, ART - zb * CH))
    for s in dumps:
        pltpu.async_copy(acc.at[s], p_h.at[cid, s], sd)
    for s in dumps:
        pltpu.make_async_copy(acc.at[s], p_h.at[cid, s], sd).wait()


def _make_scatter_kernel(nch):
    return pl.kernel(
        functools.partial(_scatter_body, nch),
        out_type=jax.ShapeDtypeStruct((NC, NP, D), jnp.float32),
        mesh=_mesh,
        compiler_params=_params,
        scratch_types=[
            pltpu.VMEM_SHARED((AR, D), jnp.float32),
            [pltpu.VMEM((CH, D), jnp.float32) for _ in range(NB)],
            [pltpu.VMEM((NB, CH), jnp.int32) for _ in range(2)],
            [pltpu.VMEM((NB, CH), jnp.int32) for _ in range(2)],
            [pltpu.SemaphoreType.DMA for _ in range(NB)],
            [pltpu.SemaphoreType.DMA for _ in range(NB)],
            [pltpu.SemaphoreType.DMA for _ in range(2)],
            pltpu.SemaphoreType.DMA,
        ],
    )


_TBLK = 512


def _tcc1_body(p0_ref, p1_ref, g_ref, d2_ref, gn_ref):
    gn_ref[...] = d2_ref[...] * (p0_ref[0] + p1_ref[0] + g_ref[...])


def _tcc2_body(final, tk_ref, gg_ref, gn_ref, dm_ref, *out_refs):
    gv = gg_ref[...] + tk_ref[0, 0] * gn_ref[...]
    out_refs[0][...] = gv
    if final:
        out_refs[1][...] = dm_ref[...] * gv


_bs_nd = pl.BlockSpec((_TBLK, D), lambda i: (i, 0))
_bs_s = pl.BlockSpec((_TBLK, 1), lambda i: (i, 0))
_nblk = NP // _TBLK

_tc_comb1 = pl.pallas_call(
    _tcc1_body,
    grid=(_nblk,),
    in_specs=[
        pl.BlockSpec((1, _TBLK, D), lambda i: (0, i, 0)),
        pl.BlockSpec((1, _TBLK, D), lambda i: (1, i, 0)),
        _bs_nd,
        _bs_s,
    ],
    out_specs=_bs_nd,
    out_shape=jax.ShapeDtypeStruct((NP, D), jnp.float32),
)


def _make_tc_comb2(final):
    outs = [jax.ShapeDtypeStruct((NP, D), jnp.float32)]
    if final:
        outs.append(jax.ShapeDtypeStruct((NP, D), jnp.float32))
    return pl.pallas_call(
        functools.partial(_tcc2_body, final),
        grid=(_nblk,),
        in_specs=[
            pl.BlockSpec(memory_space=pltpu.SMEM),
            _bs_nd,
            _bs_nd,
            _bs_s,
        ],
        out_specs=[_bs_nd] * len(outs),
        out_shape=outs,
    )


def kernel(x, edge_index, temp):
    E = edge_index.shape[1]
    nch = -(-E // (NT * CH))
    nch = -(-nch // NB) * NB
    if (nch // NB) % 2 == 0:
        nch += NB
    ep = NT * nch * CH
    row = edge_index[0]
    col = edge_index[1]
    pad_i = jnp.arange(ep - E, dtype=jnp.int32)
    rowp = jnp.concatenate([row, pad_i % N]).reshape(NT, nch, CH)
    colp = jnp.concatenate(
        [col, N + pad_i % (AR - N)]).reshape(NT, nch, CH)
    rowp = jnp.pad(rowp, ((0, 0), (0, NB), (0, 0)))
    colp = jnp.pad(colp, ((0, 0), (0, NB), (0, 0)))
    xp = jnp.pad(x, ((0, NP - N), (0, 0)))

    deg_k = _make_deg_kernel(nch)
    scat_k = _make_scatter_kernel(nch)
    comb_fin = _make_tc_comb2(True)
    comb_mid = _make_tc_comb2(False)

    degp = deg_k(colp)
    t0b = jnp.broadcast_to(temp[0], (VL,))
    dinv2, dm, g, gg = _norm_kernel(degp, xp, t0b)
    d2c = dinv2.reshape(NP, 1)
    dmc = dm.reshape(NP, 1)
    hid = None
    for k in range(1, K + 1):
        p = scat_k(g, rowp, colp)
        g = _tc_comb1(p, p, g, d2c)
        tk = temp[k].reshape(1, 1)
        if k < K:
            (gg,) = comb_mid(tk, gg, g, dmc)
        else:
            gg, hid = comb_fin(tk, gg, g, dmc)
    return hid[:N]


# speedup vs baseline: 1.0167x; 1.0167x over previous
import functools

import jax
import jax.numpy as jnp
from jax import lax
from jax.experimental import pallas as pl
from jax.experimental.pallas import tpu as pltpu
from jax.experimental.pallas import tpu_sc as plsc

N = 10000
D = 128
K = 10
NC = 2
NS = 16
NT = NC * NS
NP = 10240
RPT = NP // NT
RPS = NP // NS
CH = 128
VL = 16

_mesh = plsc.VectorSubcoreMesh(
    core_axis_name="c", subcore_axis_name="s", num_cores=NC, num_subcores=NS)
_params = pltpu.CompilerParams(
    needs_layout_passes=False, use_tc_tiling_on_sc=False)


def _wid():
    return lax.axis_index("c") * NS + lax.axis_index("s")


def _splat_i32(i):
    return jnp.full((VL,), i, dtype=jnp.int32)


def _rsqrt(d):
    i = plsc.bitcast(d, jnp.int32)
    y = plsc.bitcast(jnp.int32(0x5F3759DF) - (i >> 1), jnp.float32)
    for _ in range(3):
        y = y * (1.5 - 0.5 * d * y * y)
    return y


def _zero_rows(buf, nrows):
    z = jnp.zeros((VL,), jnp.float32)

    def body(i, c):
        for l in range(D // VL):
            buf[i, pl.ds(l * VL, VL)] = z
        return c

    lax.fori_loop(0, nrows, body, 0)


def _deg_body(nch, colp_h, degp_h, colv, degv):
    w = _wid()
    z = jnp.zeros((VL,), jnp.float32)

    def zbody(i, c):
        degv[pl.ds(i * VL, VL)] = z
        return c

    lax.fori_loop(0, NP // VL, zbody, 0)
    pltpu.sync_copy(colp_h.at[w, pl.ds(0, nch)], colv)
    ones = jnp.ones((VL,), jnp.float32)

    def cbody(j, c):
        for l in range(CH // VL):
            idx = colv[j, pl.ds(l * VL, VL)]
            plsc.addupdate_scatter(degv, [idx], ones)
        return c

    lax.fori_loop(0, nch, cbody, 0)
    pltpu.sync_copy(degv, degp_h.at[w])


def _make_deg_kernel(nch):
    return pl.kernel(
        functools.partial(_deg_body, nch),
        out_type=jax.ShapeDtypeStruct((NT, NP), jnp.float32),
        mesh=_mesh,
        compiler_params=_params,
        scratch_types=[
            pltpu.VMEM((nch, CH), jnp.int32),
            pltpu.VMEM((NP,), jnp.float32),
        ],
    )


def _norm_body(degp_h, xp_h, temp_h, dinv2_h, dm_h, g0_h, gg0_h,
               degs, dv, d2v, dmv, xbuf, tbuf, sp):
    w = _wid()
    base = w * RPT
    for t in range(NT):
        pltpu.async_copy(degp_h.at[t, pl.ds(base, RPT)], degs.at[t], sp)
    for t in range(NT):
        pltpu.make_async_copy(
            degp_h.at[t, pl.ds(base, RPT)], degs.at[t], sp).wait()

    def rbody(i, c):
        acc = jnp.ones((VL,), jnp.float32)
        for t in range(NT):
            acc = acc + degs[t, pl.ds(i * VL, VL)]
        y = _rsqrt(acc)
        dv[pl.ds(i * VL, VL)] = y
        d2v[pl.ds(i * VL, VL)] = y * y
        dmv[pl.ds(i * VL, VL)] = acc * y
        return c

    lax.fori_loop(0, RPT // VL, rbody, 0)
    pltpu.sync_copy(d2v, dinv2_h.at[pl.ds(base, RPT)])
    pltpu.sync_copy(dmv, dm_h.at[pl.ds(base, RPT)])
    pltpu.sync_copy(temp_h, tbuf)
    t0 = tbuf[pl.ds(0, VL)]

    nr = 64
    for cc in range(RPT // nr):
        o = base + cc * nr
        pltpu.sync_copy(xp_h.at[pl.ds(o, nr)], xbuf)

        def sbody(i, c, cc=cc):
            dvi = plsc.load_gather(dv, [_splat_i32(cc * nr + i)])
            for l in range(D // VL):
                s = pl.ds(l * VL, VL)
                xbuf[i, s] = xbuf[i, s] * dvi
            return c

        lax.fori_loop(0, nr, sbody, 0)
        pltpu.sync_copy(xbuf, g0_h.at[pl.ds(o, nr)])

        def t0body(i, c):
            for l in range(D // VL):
                s = pl.ds(l * VL, VL)
                xbuf[i, s] = xbuf[i, s] * t0
            return c

        lax.fori_loop(0, nr, t0body, 0)
        pltpu.sync_copy(xbuf, gg0_h.at[pl.ds(o, nr)])


_norm_kernel = pl.kernel(
    _norm_body,
    out_type=(
        jax.ShapeDtypeStruct((NP,), jnp.float32),
        jax.ShapeDtypeStruct((NP,), jnp.float32),
        jax.ShapeDtypeStruct((NP, D), jnp.float32),
        jax.ShapeDtypeStruct((NP, D), jnp.float32),
    ),
    mesh=_mesh,
    compiler_params=_params,
    scratch_types=[
        pltpu.VMEM((NT, RPT), jnp.float32),
        pltpu.VMEM((RPT,), jnp.float32),
        pltpu.VMEM((RPT,), jnp.float32),
        pltpu.VMEM((RPT,), jnp.float32),
        pltpu.VMEM((64, D), jnp.float32),
        pltpu.VMEM((VL,), jnp.float32),
        pltpu.SemaphoreType.DMA,
    ],
)


NB = 3
AR = 10016
ART = AR // NS


def _scatter_body(nch, g_h, rowp_h, colp_h, p_h,
                  acc, rb, rix, cix, sg, ss, si, sd):
    cid = lax.axis_index("c")
    sid = lax.axis_index("s")
    w = cid * NS + sid

    def idx_start(s, grp):
        pltpu.async_copy(rowp_h.at[w, pl.ds(grp * NB, NB)], rix[s], si[s])
        pltpu.async_copy(colp_h.at[w, pl.ds(grp * NB, NB)], cix[s], si[s])

    def idx_wait(s):
        pltpu.make_async_copy(rowp_h.at[w, pl.ds(0, NB)], rix[s], si[s]).wait()
        pltpu.make_async_copy(colp_h.at[w, pl.ds(0, NB)], cix[s], si[s]).wait()

    def gwait(b):
        pltpu.make_async_copy(g_h.at[pl.ds(0, CH)], rb[b], sg[b]).wait()

    def swait(b):
        pltpu.make_async_copy(g_h.at[pl.ds(0, CH)], rb[b], ss[b]).wait()

    ngrp = nch // NB
    idx_start(0, 0)
    _zero_rows(rb[0], CH)
    zb = ART // CH
    for z in range(zb):
        pltpu.sync_copy(rb[0], acc.at[pl.ds(sid * ART + z * CH, CH)])
    pltpu.sync_copy(rb[0].at[pl.ds(0, ART - zb * CH)],
                    acc.at[pl.ds(sid * ART + zb * CH, ART - zb * CH)])
    idx_wait(0)
    for b in range(NB):
        pltpu.async_copy(g_h.at[rix[0].at[b]], rb[b], sg[b])
    idx_start(1, 1)
    plsc.subcore_barrier()

    def group(gg, s_cur, s_nxt):
        for b in range(NB):
            gwait(b)
            pltpu.async_copy(rb[b], acc.at[cix[s_cur].at[b]], ss[b], add=True)
        idx_wait(s_nxt)
        for b in range(NB):
            swait(b)
            pltpu.async_copy(g_h.at[rix[s_nxt].at[b]], rb[b], sg[b])
        idx_start(s_cur, gg + 2)

    def ebody(m2, c):
        group(2 * m2, 0, 1)
        group(2 * m2 + 1, 1, 0)
        return c

    lax.fori_loop(0, (ngrp - 1) // 2, ebody, 0)
    idx_wait(1)
    for b in range(NB):
        gwait(b)
        pltpu.async_copy(rb[b], acc.at[cix[0].at[b]], ss[b], add=True)
    for b in range(NB):
        swait(b)
    plsc.subcore_barrier()
    dumps = [pl.ds(sid * ART + z * CH, CH) for z in range(zb)]
    dumps.append(pl.ds(sid * ART + zb * CH, ART - zb * CH))
    for s in dumps:
        pltpu.async_copy(acc.at[s], p_h.at[cid, s], sd)
    for s in dumps:
        pltpu.make_async_copy(acc.at[s], p_h.at[cid, s], sd).wait()


def _make_scatter_kernel(nch):
    return pl.kernel(
        functools.partial(_scatter_body, nch),
        out_type=jax.ShapeDtypeStruct((NC, NP, D), jnp.float32),
        mesh=_mesh,
        compiler_params=_params,
        scratch_types=[
            pltpu.VMEM_SHARED((AR, D), jnp.float32),
            [pltpu.VMEM((CH, D), jnp.float32) for _ in range(NB)],
            [pltpu.VMEM((NB, CH), jnp.int32) for _ in range(2)],
            [pltpu.VMEM((NB, CH), jnp.int32) for _ in range(2)],
            [pltpu.SemaphoreType.DMA for _ in range(NB)],
            [pltpu.SemaphoreType.DMA for _ in range(NB)],
            [pltpu.SemaphoreType.DMA for _ in range(2)],
            pltpu.SemaphoreType.DMA,
        ],
    )


_TBLK = 512


def _tcc_body(final, tk_ref, p0_ref, p1_ref, g_ref, gg_ref, d2_ref, dm_ref,
              *out_refs):
    d2 = d2_ref[...]
    gp = d2 * (p0_ref[0] + p1_ref[0] + g_ref[...])
    out_refs[0][...] = gp
    gv = gg_ref[...] + tk_ref[0, 0] * gp
    out_refs[1][...] = gv
    if final:
        out_refs[2][...] = dm_ref[...] * gv


def _make_tc_combine(final):
    outs = [
        jax.ShapeDtypeStruct((NP, D), jnp.float32),
        jax.ShapeDtypeStruct((NP, D), jnp.float32),
    ]
    if final:
        outs.append(jax.ShapeDtypeStruct((NP, D), jnp.float32))
    nblk = NP // _TBLK
    bs_nd = pl.BlockSpec((_TBLK, D), lambda i: (i, 0))
    bs_s = pl.BlockSpec((_TBLK, 1), lambda i: (i, 0))
    return pl.pallas_call(
        functools.partial(_tcc_body, final),
        grid=(nblk,),
        in_specs=[
            pl.BlockSpec(memory_space=pltpu.SMEM),
            pl.BlockSpec((1, _TBLK, D), lambda i: (0, i, 0)),
            pl.BlockSpec((1, _TBLK, D), lambda i: (1, i, 0)),
            bs_nd,
            bs_nd,
            bs_s,
            bs_s,
        ],
        out_specs=[bs_nd] * len(outs),
        out_shape=outs,
    )


def kernel(x, edge_index, temp):
    E = edge_index.shape[1]
    nch = -(-E // (NT * CH))
    nch = -(-nch // NB) * NB
    if (nch // NB) % 2 == 0:
        nch += NB
    ep = NT * nch * CH
    row = edge_index[0]
    col = edge_index[1]
    pad_i = jnp.arange(ep - E, dtype=jnp.int32)
    rowp = jnp.concatenate([row, pad_i % N]).reshape(NT, nch, CH)
    colp = jnp.concatenate(
        [col, N + pad_i % (AR - N)]).reshape(NT, nch, CH)
    rowp = jnp.pad(rowp, ((0, 0), (0, NB), (0, 0)))
    colp = jnp.pad(colp, ((0, 0), (0, NB), (0, 0)))
    xp = jnp.pad(x, ((0, NP - N), (0, 0)))

    deg_k = _make_deg_kernel(nch)
    scat_k = _make_scatter_kernel(nch)
    comb_mid = _make_tc_combine(False)
    comb_fin = _make_tc_combine(True)

    degp = deg_k(colp)
    t0b = jnp.broadcast_to(temp[0], (VL,))
    dinv2, dm, g, gg = _norm_kernel(degp, xp, t0b)
    d2c = dinv2.reshape(NP, 1)
    dmc = dm.reshape(NP, 1)
    hid = None
    for k in range(1, K + 1):
        p = scat_k(g, rowp, colp)
        tk = temp[k].reshape(1, 1)
        if k < K:
            g, gg = comb_mid(tk, p, p, g, gg, d2c, dmc)
        else:
            g, gg, hid = comb_fin(tk, p, p, g, gg, d2c, dmc)
    return hid[:N]
